# Initial kernel scaffold; baseline (speedup 1.0000x reference)
import jax
import jax.numpy as jnp
from jax import lax
from jax.experimental import pallas as pl
from jax.experimental.pallas import tpu as pltpu
from jax.experimental.pallas import tpu_sc as plsc

D = 128
DP = 144
NC = 2
NS = 16
L = 16
SUB = 128
NSUB = 4
CH = SUB * NSUB


def _sc_body_maker(n_pad, nchunk, rows_per_tile):
    stripe = n_pad // NS

    def body(feat_hbm, src_hbm, dst_hbm, zeros_hbm, pacc_hbm,
             src_v, dst_v, rows_v, z_v, acc_sh, sem):
        c = lax.axis_index("c")
        s = lax.axis_index("s")
        tid = c * NS + s

        pltpu.sync_copy(zeros_hbm, z_v)

        @pl.loop(0, stripe // L)
        def _zero(k):
            pltpu.sync_copy(z_v, acc_sh.at[pl.ds(s * stripe + k * L, L)])

        plsc.subcore_barrier()

        row_base = tid * rows_per_tile

        @pl.loop(0, nchunk)
        def _chunk(i):
            r0 = row_base + i * NSUB
            pltpu.sync_copy(src_hbm.at[pl.ds(r0, NSUB)], src_v)
            pltpu.sync_copy(dst_hbm.at[pl.ds(r0, NSUB)], dst_v)
            cps = [
                pltpu.async_copy(feat_hbm.at[src_v.at[j]],
                                 rows_v.at[pl.ds(j * SUB, SUB)], sem)
                for j in range(NSUB)
            ]
            for cp in cps:
                cp.wait()
            for j in range(NSUB):
                pltpu.sync_copy(rows_v.at[pl.ds(j * SUB, SUB)],
                                acc_sh.at[dst_v.at[j]], add=True)

        plsc.subcore_barrier()
        pltpu.sync_copy(acc_sh.at[pl.ds(s * stripe, stripe)],
                        pacc_hbm.at[pl.ds((c * n_pad + s * stripe), stripe)])

    return body


def _tc_body(feat_ref, a0_ref, a1_ref, ws_ref, wn_ref, bs_ref, bn_ref,
             out_ref):
    x = feat_ref[...]
    a = a0_ref[...] + a1_ref[...]
    ssum = a[:, :D]
    cnt = a[:, D:D + 1]
    h = ssum / jnp.maximum(cnt, 1.0)
    out_ref[...] = (
        jnp.dot(x, ws_ref[...], preferred_element_type=jnp.float32)
        + jnp.dot(h, wn_ref[...], preferred_element_type=jnp.float32)
        + bs_ref[...] + bn_ref[...]
    )


def kernel(feat, edge_index, W_self, b_self, W_neigh, b_neigh):
    n, d = feat.shape
    e = edge_index.shape[1]
    assert d == D

    n_pad = -(-n // (NS * L)) * (NS * L)
    tile_e = -(-e // (NC * NS * CH)) * CH
    e_pad = tile_e * NC * NS
    nchunk = tile_e // CH
    rows_per_tile = tile_e // SUB
    blk = 512
    n_blocks = n_pad // blk

    src = edge_index[0]
    dst = edge_index[1]
    pad_e = e_pad - e
    src_p = jnp.concatenate(
        [src, jnp.zeros((pad_e,), jnp.int32)]).reshape(e_pad // SUB, SUB)
    dst_p = jnp.concatenate(
        [dst, jnp.full((pad_e,), n_pad - 1, jnp.int32)]
    ).reshape(e_pad // SUB, SUB)
    featx = jnp.pad(feat, ((0, 0), (0, DP - D)), constant_values=1.0)
    zeros_blk = jnp.zeros((L, DP), jnp.float32)

    mesh = plsc.VectorSubcoreMesh(core_axis_name="c", subcore_axis_name="s")
    sc_call = pl.kernel(
        _sc_body_maker(n_pad, nchunk, rows_per_tile),
        out_type=jax.ShapeDtypeStruct((NC * n_pad, DP), jnp.float32),
        mesh=mesh,
        scratch_types=[
            pltpu.VMEM((NSUB, SUB), jnp.int32),
            pltpu.VMEM((NSUB, SUB), jnp.int32),
            pltpu.VMEM((CH, DP), jnp.float32),
            pltpu.VMEM((L, DP), jnp.float32),
            pltpu.VMEM_SHARED((n_pad, DP), jnp.float32),
            pltpu.SemaphoreType.DMA,
        ],
        name="sage_sc_aggregate",
    )
    pacc = sc_call(featx, src_p, dst_p, zeros_blk)

    feat_pad = jnp.pad(feat, ((0, n_pad - n), (0, 0)))
    out = pl.pallas_call(
        _tc_body,
        grid=(n_blocks,),
        in_specs=[
            pl.BlockSpec((blk, D), lambda i: (i, 0)),
            pl.BlockSpec((blk, DP), lambda i: (i, 0)),
            pl.BlockSpec((blk, DP), lambda i: (n_blocks + i, 0)),
            pl.BlockSpec((D, D), lambda i: (0, 0)),
            pl.BlockSpec((D, D), lambda i: (0, 0)),
            pl.BlockSpec((1, D), lambda i: (0, 0)),
            pl.BlockSpec((1, D), lambda i: (0, 0)),
        ],
        out_specs=pl.BlockSpec((blk, D), lambda i: (i, 0)),
        out_shape=jax.ShapeDtypeStruct((n_pad, D), jnp.float32),
        name="sage_tc_combine",
    )(feat_pad, pacc, pacc, W_self.T, W_neigh.T,
      b_self.reshape(1, D), b_neigh.reshape(1, D))
    return out[:n]


# speedup vs baseline: 3.0257x; 3.0257x over previous
import jax
import jax.numpy as jnp
from jax import lax
from jax.experimental import pallas as pl
from jax.experimental.pallas import tpu as pltpu
from jax.experimental.pallas import tpu_sc as plsc

D = 128
DP = 144
NC = 2
NS = 16
L = 16
SUB = 128
NSUB = 2
CH = SUB * NSUB


def _sc_body_maker(n_pad, nchunk, rows_per_tile):
    stripe = n_pad // NS

    def body(feat_hbm, src_hbm, dst_hbm, zeros_hbm, pacc_hbm,
             src_v, dst_v, rows_v, acc_sh, sem):
        c = lax.axis_index("c")
        s = lax.axis_index("s")
        tid = c * NS + s

        pltpu.sync_copy(zeros_hbm, rows_v.at[pl.ds(0, L)])

        @pl.loop(0, stripe // L)
        def _zero(k):
            pltpu.sync_copy(rows_v.at[pl.ds(0, L)],
                            acc_sh.at[pl.ds(s * stripe + k * L, L)])

        plsc.subcore_barrier()

        row_base = tid * rows_per_tile

        @pl.loop(0, nchunk)
        def _chunk(i):
            r0 = row_base + i * NSUB
            pltpu.sync_copy(src_hbm.at[pl.ds(r0, NSUB)], src_v)
            pltpu.sync_copy(dst_hbm.at[pl.ds(r0, NSUB)], dst_v)
            cps = [
                pltpu.async_copy(feat_hbm.at[src_v.at[j]],
                                 rows_v.at[pl.ds(j * SUB, SUB)], sem)
                for j in range(NSUB)
            ]
            for cp in cps:
                cp.wait()
            for j in range(NSUB):
                pltpu.sync_copy(rows_v.at[pl.ds(j * SUB, SUB)],
                                acc_sh.at[dst_v.at[j]], add=True)

        plsc.subcore_barrier()
        pltpu.sync_copy(acc_sh.at[pl.ds(s * stripe, stripe)],
                        pacc_hbm.at[pl.ds((c * n_pad + s * stripe), stripe)])

    return body


def _tc_body(feat_ref, a0_ref, a1_ref, ws_ref, wn_ref, bs_ref, bn_ref,
             out_ref):
    x = feat_ref[...]
    a = a0_ref[...] + a1_ref[...]
    ssum = a[:, :D]
    cnt = a[:, D:D + 1]
    h = ssum / jnp.maximum(cnt, 1.0)
    out_ref[...] = (
        jnp.dot(x, ws_ref[...], preferred_element_type=jnp.float32)
        + jnp.dot(h, wn_ref[...], preferred_element_type=jnp.float32)
        + bs_ref[...] + bn_ref[...]
    )


def kernel(feat, edge_index, W_self, b_self, W_neigh, b_neigh):
    n, d = feat.shape
    e = edge_index.shape[1]
    assert d == D

    n_pad = -(-n // (NS * L)) * (NS * L)
    tile_e = -(-e // (NC * NS * CH)) * CH
    e_pad = tile_e * NC * NS
    nchunk = tile_e // CH
    rows_per_tile = tile_e // SUB
    blk = 512
    n_blocks = n_pad // blk

    src = edge_index[0]
    dst = edge_index[1]
    pad_e = e_pad - e
    src_p = jnp.concatenate(
        [src, jnp.zeros((pad_e,), jnp.int32)]).reshape(e_pad // SUB, SUB)
    dst_p = jnp.concatenate(
        [dst, jnp.full((pad_e,), n_pad - 1, jnp.int32)]
    ).reshape(e_pad // SUB, SUB)
    featx = jnp.pad(feat, ((0, 0), (0, DP - D)), constant_values=1.0)
    zeros_blk = jnp.zeros((L, DP), jnp.float32)

    mesh = plsc.VectorSubcoreMesh(core_axis_name="c", subcore_axis_name="s")
    sc_call = pl.kernel(
        _sc_body_maker(n_pad, nchunk, rows_per_tile),
        out_type=jax.ShapeDtypeStruct((NC * n_pad, DP), jnp.float32),
        mesh=mesh,
        scratch_types=[
            pltpu.VMEM((NSUB, SUB), jnp.int32),
            pltpu.VMEM((NSUB, SUB), jnp.int32),
            pltpu.VMEM((CH, DP), jnp.float32),
            pltpu.VMEM_SHARED((n_pad, DP), jnp.float32),
            pltpu.SemaphoreType.DMA,
        ],
        compiler_params=pltpu.CompilerParams(use_tc_tiling_on_sc=False),
        name="sage_sc_aggregate",
    )
    pacc = sc_call(featx, src_p, dst_p, zeros_blk)

    feat_pad = jnp.pad(feat, ((0, n_pad - n), (0, 0)))
    out = pl.pallas_call(
        _tc_body,
        grid=(n_blocks,),
        in_specs=[
            pl.BlockSpec((blk, D), lambda i: (i, 0)),
            pl.BlockSpec((blk, DP), lambda i: (i, 0)),
            pl.BlockSpec((blk, DP), lambda i: (n_blocks + i, 0)),
            pl.BlockSpec((D, D), lambda i: (0, 0)),
            pl.BlockSpec((D, D), lambda i: (0, 0)),
            pl.BlockSpec((1, D), lambda i: (0, 0)),
            pl.BlockSpec((1, D), lambda i: (0, 0)),
        ],
        out_specs=pl.BlockSpec((blk, D), lambda i: (i, 0)),
        out_shape=jax.ShapeDtypeStruct((n_pad, D), jnp.float32),
        name="sage_tc_combine",
    )(feat_pad, pacc, pacc, W_self.T, W_neigh.T,
      b_self.reshape(1, D), b_neigh.reshape(1, D))
    return out[:n]


# speedup vs baseline: 3.6017x; 1.1904x over previous
import jax
import jax.numpy as jnp
from jax import lax
from jax.experimental import pallas as pl
from jax.experimental.pallas import tpu as pltpu
from jax.experimental.pallas import tpu_sc as plsc

D = 128
DP = 144
NC = 2
NS = 16
L = 16
SUB = 128
NSUB = 2
CH = SUB * NSUB


def _sc_body_maker(n_pad, nchunk, rows_per_tile):
    stripe = n_pad // NS

    def body(feat_hbm, src_hbm, dst_hbm, zeros_hbm, pacc_hbm,
             src_v, dst_v, rows_v, acc_sh, sem, isem0, isem1):
        c = lax.axis_index("c")
        s = lax.axis_index("s")
        tid = c * NS + s
        isems = (isem0, isem1)

        pltpu.sync_copy(zeros_hbm, rows_v.at[pl.ds(0, L)])

        @pl.loop(0, stripe // L)
        def _zero(k):
            pltpu.sync_copy(rows_v.at[pl.ds(0, L)],
                            acc_sh.at[pl.ds(s * stripe + k * L, L)])

        plsc.subcore_barrier()

        row_base = tid * rows_per_tile

        def idx_start(p, r0):
            pltpu.async_copy(src_hbm.at[pl.ds(r0, NSUB)], src_v.at[p],
                             isems[p])
            pltpu.async_copy(dst_hbm.at[pl.ds(r0, NSUB)], dst_v.at[p],
                             isems[p])

        def idx_wait(p):
            pltpu.make_async_copy(src_hbm.at[pl.ds(row_base, NSUB)],
                                  src_v.at[p], isems[p]).wait()
            pltpu.make_async_copy(dst_hbm.at[pl.ds(row_base, NSUB)],
                                  dst_v.at[p], isems[p]).wait()

        def chunk(p, r0):
            idx_wait(p)
            cps = [
                pltpu.async_copy(feat_hbm.at[src_v.at[p, j]],
                                 rows_v.at[pl.ds(j * SUB, SUB)], sem)
                for j in range(NSUB)
            ]
            idx_start(1 - p, r0 + NSUB)
            for j in range(NSUB):
                cps[j].wait()
                pltpu.sync_copy(rows_v.at[pl.ds(j * SUB, SUB)],
                                acc_sh.at[dst_v.at[p, j]], add=True)

        idx_start(0, row_base)

        @pl.loop(0, nchunk // 2)
        def _chunk(i):
            r0 = row_base + i * 2 * NSUB
            chunk(0, r0)
            chunk(1, r0 + NSUB)

        idx_wait(0)

        plsc.subcore_barrier()
        pltpu.sync_copy(acc_sh.at[pl.ds(s * stripe, stripe)],
                        pacc_hbm.at[pl.ds((c * n_pad + s * stripe), stripe)])

    return body


def _tc_body(feat_ref, a0_ref, a1_ref, ws_ref, wn_ref, bs_ref, bn_ref,
             out_ref):
    x = feat_ref[...]
    a = a0_ref[...] + a1_ref[...]
    ssum = a[:, :D]
    cnt = a[:, D:D + 1]
    h = ssum / jnp.maximum(cnt, 1.0)
    out_ref[...] = (
        jnp.dot(x, ws_ref[...], preferred_element_type=jnp.float32)
        + jnp.dot(h, wn_ref[...], preferred_element_type=jnp.float32)
        + bs_ref[...] + bn_ref[...]
    )


def kernel(feat, edge_index, W_self, b_self, W_neigh, b_neigh):
    n, d = feat.shape
    e = edge_index.shape[1]
    assert d == D

    n_pad = -(-n // (NS * L)) * (NS * L)
    tile_e = -(-e // (NC * NS * CH)) * CH
    e_pad = tile_e * NC * NS
    nchunk = tile_e // CH
    rows_per_tile = tile_e // SUB
    blk = 512
    n_blocks = n_pad // blk

    src = edge_index[0]
    dst = edge_index[1]
    pad_e = e_pad - e
    src_p = jnp.pad(jnp.concatenate(
        [src, jnp.zeros((pad_e,), jnp.int32)]).reshape(e_pad // SUB, SUB),
        ((0, NSUB), (0, 0)))
    dst_p = jnp.pad(jnp.concatenate(
        [dst, jnp.full((pad_e,), n_pad - 1, jnp.int32)]
    ).reshape(e_pad // SUB, SUB), ((0, NSUB), (0, 0)))
    featx = jnp.pad(feat, ((0, 0), (0, DP - D)), constant_values=1.0)
    zeros_blk = jnp.zeros((L, DP), jnp.float32)

    mesh = plsc.VectorSubcoreMesh(core_axis_name="c", subcore_axis_name="s")
    sc_call = pl.kernel(
        _sc_body_maker(n_pad, nchunk, rows_per_tile),
        out_type=jax.ShapeDtypeStruct((NC * n_pad, DP), jnp.float32),
        mesh=mesh,
        scratch_types=[
            pltpu.VMEM((2, NSUB, SUB), jnp.int32),
            pltpu.VMEM((2, NSUB, SUB), jnp.int32),
            pltpu.VMEM((CH, DP), jnp.float32),
            pltpu.VMEM_SHARED((n_pad, DP), jnp.float32),
            pltpu.SemaphoreType.DMA,
            pltpu.SemaphoreType.DMA,
            pltpu.SemaphoreType.DMA,
        ],
        compiler_params=pltpu.CompilerParams(use_tc_tiling_on_sc=False),
        name="sage_sc_aggregate",
    )
    pacc = sc_call(featx, src_p, dst_p, zeros_blk)

    feat_pad = jnp.pad(feat, ((0, n_pad - n), (0, 0)))
    out = pl.pallas_call(
        _tc_body,
        grid=(n_blocks,),
        in_specs=[
            pl.BlockSpec((blk, D), lambda i: (i, 0)),
            pl.BlockSpec((blk, DP), lambda i: (i, 0)),
            pl.BlockSpec((blk, DP), lambda i: (n_blocks + i, 0)),
            pl.BlockSpec((D, D), lambda i: (0, 0)),
            pl.BlockSpec((D, D), lambda i: (0, 0)),
            pl.BlockSpec((1, D), lambda i: (0, 0)),
            pl.BlockSpec((1, D), lambda i: (0, 0)),
        ],
        out_specs=pl.BlockSpec((blk, D), lambda i: (i, 0)),
        out_shape=jax.ShapeDtypeStruct((n_pad, D), jnp.float32),
        name="sage_tc_combine",
    )(feat_pad, pacc, pacc, W_self.T, W_neigh.T,
      b_self.reshape(1, D), b_neigh.reshape(1, D))
    return out[:n]


# speedup vs baseline: 6.3382x; 1.7598x over previous
import jax
import jax.numpy as jnp
from jax import lax
from jax.experimental import pallas as pl
from jax.experimental.pallas import tpu as pltpu
from jax.experimental.pallas import tpu_sc as plsc

D = 128
DP = 144
NC = 2
NS = 16
L = 16
SUB = 128


def _sc_body_maker(n_pad, units):
    stripe = n_pad // NS

    def body(feat_hbm, edges_hbm, zeros_hbm, pacc_hbm,
             idx_v, rows0, rows1, acc_sh,
             gsem0, gsem1, isem0, isem1, isem2, isem3):
        c = lax.axis_index("c")
        s = lax.axis_index("s")
        tid = c * NS + s
        rows = (rows0, rows1)
        gsems = (gsem0, gsem1)
        isems = (isem0, isem1, isem2, isem3)

        pltpu.sync_copy(zeros_hbm, rows0.at[pl.ds(0, L)])

        @pl.loop(0, stripe // L)
        def _zero(k):
            pltpu.sync_copy(rows0.at[pl.ds(0, L)],
                            acc_sh.at[pl.ds(s * stripe + k * L, L)])

        plsc.subcore_barrier()

        row_base = tid * units

        g_desc = {}
        i_desc = {}

        def idx_start(k):
            i_desc[k] = pltpu.async_copy(
                edges_hbm.at[row_base + k], idx_v.at[k % 4], isems[k % 4])

        def gather_start(k):
            g_desc[k] = pltpu.async_copy(
                feat_hbm.at[idx_v.at[k % 4, 0]], rows[k % 2],
                gsems[k % 2])

        def scatter_sync(k):
            pltpu.sync_copy(rows[k % 2], acc_sh.at[idx_v.at[k % 4, 1]],
                            add=True)

        idx_start(0)
        i_desc[0].wait()
        idx_start(1)
        idx_start(2)
        gather_start(0)
        for k in range(units):
            g_desc[k].wait()
            if k + 3 < units:
                idx_start(k + 3)
            if k + 1 < units:
                i_desc[k + 1].wait()
                gather_start(k + 1)
            scatter_sync(k)

        plsc.subcore_barrier()
        pltpu.sync_copy(acc_sh.at[pl.ds(s * stripe, stripe)],
                        pacc_hbm.at[c, pl.ds(s * stripe, stripe)])

    return body


def _tc_body(feat_ref, pacc_ref, ws_ref, wn_ref, bs_ref, bn_ref,
             out_ref):
    x = feat_ref[...]
    a = pacc_ref[0] + pacc_ref[1]
    ssum = a[:, :D]
    cnt = a[:, D:D + 1]
    h = ssum / jnp.maximum(cnt, 1.0)
    out_ref[...] = (
        jnp.dot(x, ws_ref[...], preferred_element_type=jnp.float32)
        + jnp.dot(h, wn_ref[...], preferred_element_type=jnp.float32)
        + bs_ref[...] + bn_ref[...]
    )


def kernel(feat, edge_index, W_self, b_self, W_neigh, b_neigh):
    n, d = feat.shape
    e = edge_index.shape[1]
    assert d == D

    n_pad = -(-n // (NS * L)) * (NS * L)
    tile_e = -(-e // (NC * NS * SUB)) * SUB
    e_pad = tile_e * NC * NS
    units = tile_e // SUB
    blk = 400
    n_blocks = n // blk

    src = edge_index[0]
    dst = edge_index[1]
    pad_e = e_pad - e
    src_p = jnp.concatenate(
        [src, jnp.zeros((pad_e,), jnp.int32)]).reshape(e_pad // SUB, SUB)
    dst_p = jnp.concatenate(
        [dst, jnp.full((pad_e,), n_pad - 1, jnp.int32)]
    ).reshape(e_pad // SUB, SUB)
    edges3 = jnp.stack([src_p, dst_p], axis=1)
    featx = jnp.pad(feat, ((0, 0), (0, DP - D)), constant_values=1.0)
    zeros_blk = jnp.zeros((L, DP), jnp.float32)

    mesh = plsc.VectorSubcoreMesh(core_axis_name="c", subcore_axis_name="s")
    sc_call = pl.kernel(
        _sc_body_maker(n_pad, units),
        out_type=jax.ShapeDtypeStruct((NC, n_pad, DP), jnp.float32),
        mesh=mesh,
        scratch_types=[
            pltpu.VMEM((4, 2, SUB), jnp.int32),
            pltpu.VMEM((SUB, DP), jnp.float32),
            pltpu.VMEM((SUB, DP), jnp.float32),
            pltpu.VMEM_SHARED((n_pad, DP), jnp.float32),
        ] + [pltpu.SemaphoreType.DMA] * 6,
        compiler_params=pltpu.CompilerParams(use_tc_tiling_on_sc=False),
        name="sage_sc_aggregate",
    )
    pacc = sc_call(featx, edges3, zeros_blk)

    out = pl.pallas_call(
        _tc_body,
        grid=(n_blocks,),
        in_specs=[
            pl.BlockSpec((blk, D), lambda i: (i, 0)),
            pl.BlockSpec((NC, blk, DP), lambda i: (0, i, 0)),
            pl.BlockSpec((D, D), lambda i: (0, 0)),
            pl.BlockSpec((D, D), lambda i: (0, 0)),
            pl.BlockSpec((1, D), lambda i: (0, 0)),
            pl.BlockSpec((1, D), lambda i: (0, 0)),
        ],
        out_specs=pl.BlockSpec((blk, D), lambda i: (i, 0)),
        out_shape=jax.ShapeDtypeStruct((n, D), jnp.float32),
        name="sage_tc_combine",
    )(feat, pacc, W_self.T, W_neigh.T,
      b_self.reshape(1, D), b_neigh.reshape(1, D))
    return out
